# baseline (speedup 1.0000x reference)
import functools

import jax
import jax.numpy as jnp
from jax import lax
from jax.experimental import pallas as pl
from jax.experimental.pallas import tpu as pltpu
from jax.experimental.pallas import tpu_sc as plsc

_T = 16384
_D = 2048
_LANES = 16
_NC = 2
_NS = 16
_NW = _NC * _NS
_ROWS_PER_W = _T // _NW
_CHUNK = 8
_NCHUNK = _ROWS_PER_W // _CHUNK
_NBUF = 4
_K = 4


def _make_kernel():
  mesh = plsc.VectorSubcoreMesh(core_axis_name="c", subcore_axis_name="s")

  @functools.partial(
      pl.kernel,
      mesh=mesh,
      out_type=jax.ShapeDtypeStruct((_T, _D), jnp.float32),
      scratch_types=(
          [pltpu.VMEM((_CHUNK, _D), jnp.float32)] * _NBUF
          + [pltpu.VMEM((1, _D), jnp.float32), pltpu.VMEM((1,), jnp.int32)]
          + [pltpu.VMEM_SHARED((_NS, 2, _CHUNK, _D), jnp.float32)]
          + [pltpu.SemaphoreType.DMA] * (_NBUF + 2)
      ),
  )
  def add_embed(x_hbm, idx_hbm, emb_hbm, out_hbm, *refs):
    bufs = refs[:_NBUF]
    emb_v = refs[_NBUF]
    idx_v = refs[_NBUF + 1]
    shared = refs[_NBUF + 2]
    isems = refs[_NBUF + 3:_NBUF + 3 + _NBUF]
    osems = refs[_NBUF + 3 + _NBUF:]

    cid = lax.axis_index("c")
    sid = lax.axis_index("s")
    wid = sid * _NC + cid
    base = wid * _ROWS_PER_W

    pltpu.sync_copy(idx_hbm, idx_v)
    pltpu.async_copy(emb_hbm.at[idx_v], emb_v, osems[0]).wait()

    def start_in(ch, b):
      pltpu.async_copy(
          x_hbm.at[pl.ds(base + ch * _CHUNK, _CHUNK)], bufs[b], isems[b])

    for ch in range(_K):
      start_in(ch, ch % _NBUF)

    def outer(i, _):
      c = i * _NBUF
      for b in range(_NBUF):
        ch = c + b
        sl = b % 2

        pltpu.make_async_copy(
            x_hbm.at[pl.ds(0, _CHUNK)], bufs[b], isems[b]).wait()

        def col_body(j, _):
          col = pl.multiple_of(j * _LANES, _LANES)
          ev = emb_v[0, pl.ds(col, _LANES)]
          for r in range(_CHUNK):
            plsc.addupdate(bufs[b].at[r, pl.ds(col, _LANES)], ev)
          return 0

        lax.fori_loop(0, _D // _LANES, col_body, 0)

        @pl.when(ch >= 2)
        def _():
          pltpu.make_async_copy(
              shared.at[sid, sl], out_hbm.at[pl.ds(0, _CHUNK)],
              osems[sl]).wait()

        pltpu.sync_copy(bufs[b], shared.at[sid, sl])
        pltpu.async_copy(
            shared.at[sid, sl], out_hbm.at[pl.ds(base + ch * _CHUNK, _CHUNK)],
            osems[sl])

        @pl.when(ch + _K < _NCHUNK)
        def _():
          start_in(ch + _K, (b + _K) % _NBUF)

      return 0

    lax.fori_loop(0, _NCHUNK // _NBUF, outer, 0)

    for sl in range(2):
      pltpu.make_async_copy(
          shared.at[sid, sl], out_hbm.at[pl.ds(0, _CHUNK)], osems[sl]).wait()

  return add_embed


_add_embed_call = _make_kernel()


@jax.jit
def kernel(input_features, modality_indices, embedding_weight):
  out = _add_embed_call(
      input_features, modality_indices.astype(jnp.int32), embedding_weight
  )
  return out[None]


# speedup vs baseline: 1.1301x; 1.1301x over previous
import functools

import jax
import jax.numpy as jnp
from jax import lax
from jax.experimental import pallas as pl
from jax.experimental.pallas import tpu as pltpu
from jax.experimental.pallas import tpu_sc as plsc

_T = 16384
_D = 2048
_LANES = 16
_NC = 2
_NS = 16
_NW = _NC * _NS
_ROWS_PER_W = _T // _NW
_CHUNK = 8
_NCHUNK = _ROWS_PER_W // _CHUNK
_NBUF = 4
_K = 2


def _make_kernel():
  mesh = plsc.VectorSubcoreMesh(core_axis_name="c", subcore_axis_name="s")

  @functools.partial(
      pl.kernel,
      mesh=mesh,
      out_type=jax.ShapeDtypeStruct((_T, _D), jnp.float32),
      scratch_types=(
          [pltpu.VMEM((_CHUNK, _D), jnp.float32)] * _NBUF
          + [pltpu.VMEM((1, _D), jnp.float32), pltpu.VMEM((1,), jnp.int32)]
          + [pltpu.SemaphoreType.DMA] * (2 * _NBUF)
      ),
  )
  def add_embed(x_hbm, idx_hbm, emb_hbm, out_hbm, *refs):
    bufs = refs[:_NBUF]
    emb_v = refs[_NBUF]
    idx_v = refs[_NBUF + 1]
    isems = refs[_NBUF + 2:_NBUF + 2 + _NBUF]
    osems = refs[_NBUF + 2 + _NBUF:]

    wid = lax.axis_index("s") * _NC + lax.axis_index("c")
    base = wid * _ROWS_PER_W

    pltpu.sync_copy(idx_hbm, idx_v)
    pltpu.async_copy(emb_hbm.at[idx_v], emb_v, osems[0]).wait()

    def start_in(ch, b):
      pltpu.async_copy(
          x_hbm.at[pl.ds(base + ch * _CHUNK, _CHUNK)], bufs[b], isems[b])

    for ch in range(_K):
      start_in(ch, ch % _NBUF)

    def outer(i, _):
      c = i * _NBUF
      for b in range(_NBUF):
        ch = c + b

        t = ch + _K
        bt = (b + _K) % _NBUF

        @pl.when(t < _NCHUNK)
        def _():
          @pl.when(t >= _NBUF)
          def _():
            pltpu.make_async_copy(
                bufs[bt], out_hbm.at[pl.ds(0, _CHUNK)], osems[bt]).wait()

          start_in(t, bt)

        pltpu.make_async_copy(
            x_hbm.at[pl.ds(0, _CHUNK)], bufs[b], isems[b]).wait()

        def col_body(j, _):
          col = pl.multiple_of(j * _LANES, _LANES)
          ev = emb_v[0, pl.ds(col, _LANES)]
          for r in range(_CHUNK):
            plsc.addupdate(bufs[b].at[r, pl.ds(col, _LANES)], ev)
          return 0

        lax.fori_loop(0, _D // _LANES, col_body, 0)

        pltpu.async_copy(
            bufs[b], out_hbm.at[pl.ds(base + ch * _CHUNK, _CHUNK)], osems[b])

      return 0

    lax.fori_loop(0, _NCHUNK // _NBUF, outer, 0)

    for b in range(_NBUF):
      pltpu.make_async_copy(
          bufs[b], out_hbm.at[pl.ds(0, _CHUNK)], osems[b]).wait()

  return add_embed


_add_embed_call = _make_kernel()


@jax.jit
def kernel(input_features, modality_indices, embedding_weight):
  out = _add_embed_call(
      input_features, modality_indices.astype(jnp.int32), embedding_weight
  )
  return out[None]
